# baseline (speedup 1.0000x reference)
import functools

import jax
import jax.numpy as jnp
from jax.experimental import pallas as pl
from jax.experimental.pallas import tpu as pltpu

_TN = 256


def _layer_body(relu_first, w_ref, x_ref, b_ref, bias_ref, mask_ref, o_ref, u_ref):
    @pl.when(pl.program_id(1) == 0)
    def _build_u():
        xb = x_ref[0]
        for j in range(3):
            u_ref[j] = jnp.dot(xb, b_ref[j], preferred_element_type=jnp.float32)

    z = jnp.dot(w_ref[0, 0], u_ref[0], preferred_element_type=jnp.float32)
    z += jnp.dot(w_ref[0, 1], u_ref[1], preferred_element_type=jnp.float32)
    z += jnp.dot(w_ref[0, 2], u_ref[2], preferred_element_type=jnp.float32)
    z = z + bias_ref[...]
    if relu_first:
        z = jnp.concatenate([jnp.maximum(z[:, :16], 0.0), z[:, 16:]], axis=1)
    o_ref[0] = z * mask_ref[0]


def _layer(w_sep, xin, b_mat, bias, mask, relu_first):
    bs, _, n, _ = w_sep.shape
    dcur = xin.shape[-1]
    grid = (bs, n // _TN)
    return pl.pallas_call(
        functools.partial(_layer_body, relu_first),
        grid=grid,
        in_specs=[
            pl.BlockSpec((1, 3, _TN, n), lambda b, t: (b, 0, t, 0)),
            pl.BlockSpec((1, n, dcur), lambda b, t: (b, 0, 0)),
            pl.BlockSpec((3, dcur, 32), lambda b, t: (0, 0, 0)),
            pl.BlockSpec((1, 32), lambda b, t: (0, 0)),
            pl.BlockSpec((1, _TN, 1), lambda b, t: (b, t, 0)),
        ],
        out_specs=pl.BlockSpec((1, _TN, 32), lambda b, t: (b, t, 0)),
        out_shape=jax.ShapeDtypeStruct((bs, n, 32), jnp.float32),
        scratch_shapes=[pltpu.VMEM((3, n, 32), jnp.float32)],
    )(w_sep, xin, b_mat, bias, mask)


def _fold(w1, w2, dcur):
    wcat = jnp.concatenate([w1, w2], axis=0)
    return wcat.reshape(32, 3, dcur).transpose(1, 2, 0)


def kernel(W, x, mask, N_batch, fc1_w0, fc1_b0, fc2_w0, fc2_b0, fc1_w1, fc1_b1,
           fc2_w1, fc2_b1, fc1_w2, fc1_b2, fc2_w2, fc2_b2, fcl_w, fcl_b):
    w_sep = jnp.transpose(W, (0, 3, 1, 2))

    b0 = _fold(fc1_w0, fc2_w0, 8)
    b1 = _fold(fc1_w1, fc2_w1, 32)
    b2 = _fold(fc1_w2, fc2_w2, 32)
    wc3 = jnp.zeros((32, 96), jnp.float32).at[:2].set(fcl_w)
    b3 = wc3.reshape(32, 3, 32).transpose(1, 2, 0)

    bias0 = jnp.concatenate([fc1_b0, fc2_b0])[None]
    bias1 = jnp.concatenate([fc1_b1, fc2_b1])[None]
    bias2 = jnp.concatenate([fc1_b2, fc2_b2])[None]
    bias3 = jnp.zeros((1, 32), jnp.float32).at[0, :2].set(fcl_b)

    cur = x
    for b_mat, bias, relu_first in (
        (b0, bias0, True),
        (b1, bias1, True),
        (b2, bias2, True),
        (b3, bias3, False),
    ):
        cur = _layer(w_sep, cur, b_mat, bias, mask, relu_first)
    return cur[:, :, :2]
